# baseline (speedup 1.0000x reference)
import numpy as np

import jax
import jax.numpy as jnp
from jax.experimental import pallas as pl
from jax.experimental.shard_map import shard_map
from jax.sharding import Mesh, PartitionSpec as P

_N = 100000
_C = 128
_BLK = 10000
_EPS = 1e-5
_BF = jnp.bfloat16


def _elu(x):
    return jnp.where(x > 0, x, jnp.exp(x) - 1.0)


def _acc_stats(i, e32, st_ref):
    s = jnp.sum(e32, axis=0, keepdims=True)
    s2 = jnp.sum(e32 * e32, axis=0, keepdims=True)
    st = jnp.concatenate([s, s2, jnp.zeros((6, _C), jnp.float32)], axis=0)

    @pl.when(i == 0)
    def _init():
        st_ref[...] = jnp.zeros_like(st_ref)

    st_ref[...] += st


def _conv1_kernel(x_ref, w_ref, b_ref, xo_ref, eo_ref, st_ref):
    h = jnp.dot(x_ref[...].astype(_BF), w_ref[...],
                preferred_element_type=jnp.float32)
    h = h + b_ref[0:1, :]
    xo_ref[...] = h.astype(_BF)
    e = _elu(h)
    eo_ref[...] = e.astype(_BF)
    _acc_stats(pl.program_id(0), e, st_ref)


def _stats_kernel(e_ref, w_ref, b_ref, st_ref):
    h = jnp.dot(e_ref[...], w_ref[...], preferred_element_type=jnp.float32)
    h = h + b_ref[0:1, :]
    _acc_stats(pl.program_id(0), _elu(h), st_ref)


def _fused_kernel(e_ref, x_ref, wa_ref, ba_ref, wb_ref, bb_ref,
                  xo_ref, eo_ref, st_ref):
    ha = jnp.dot(e_ref[...], wa_ref[...], preferred_element_type=jnp.float32)
    ea = _elu(ha + ba_ref[0:1, :]).astype(_BF)
    hb = jnp.dot(ea, wb_ref[...], preferred_element_type=jnp.float32)
    hb = hb + bb_ref[0:1, :] + x_ref[...]
    xo_ref[...] = hb.astype(_BF)
    e = _elu(hb)
    eo_ref[...] = e.astype(_BF)
    _acc_stats(pl.program_id(0), e, st_ref)


def _final_kernel(e_ref, r_ref, w_ref, b_ref, y_ref):
    h = jnp.dot(e_ref[...], w_ref[...], preferred_element_type=jnp.float32)
    y_ref[...] = h + b_ref[0:1, :] + r_ref[...]


_BIG = pl.BlockSpec((_BLK, _C), lambda i: (i, 0))
_WSP = pl.BlockSpec((_C, _C), lambda i: (0, 0))
_SML = pl.BlockSpec((8, _C), lambda i: (0, 0))


def _call(kfn, ops, in_specs, outs, n):
    out_specs = [s for s, _ in outs]
    out_shape = [jax.ShapeDtypeStruct((n, _C) if sh is None else sh, dt)
                 for _, (sh, dt) in outs]
    return pl.pallas_call(kfn, grid=(n // _BLK,), in_specs=in_specs,
                          out_specs=out_specs, out_shape=out_shape)(*ops)


def _b8(b):
    return jnp.broadcast_to(b[None, :], (8, _C))


def _fold(st, gamma_lo, beta_lo, w_lo, extra_b):
    m = st[0] * (1.0 / _N)
    v = st[1] * (1.0 / _N) - m * m
    scale = gamma_lo * jax.lax.rsqrt(v + _EPS)
    shift = beta_lo - m * scale
    return (scale[:, None] * w_lo).astype(_BF), shift @ w_lo + extra_b


_XE_ST = [(_BIG, (None, _BF)), (_BIG, (None, _BF)),
          (_SML, ((8, _C), jnp.float32))]


def _chain(x0f, W1, b1, rn, gamma2, beta2, W2, b2, n, axis):
    def allr(st):
        return jax.lax.psum(st, axis) if axis is not None else st

    x, e, st = _call(_conv1_kernel, [x0f, W1.astype(_BF), _b8(b1)],
                     [_BIG, _WSP, _SML], _XE_ST, n)
    st = allr(st)
    for i in range(4):
        ws = []
        for h in range(2):
            g = rn['gamma%d' % h][i]
            bt = rn['beta%d' % h][i]
            W = rn['W%d' % h][i]
            bb = rn['b%d' % h][i]
            ws.append((g[:_C], bt[:_C], W[:_C], bt[_C:] @ W[_C:] + bb))
        wa, ba = _fold(st, *ws[0])
        (st_a,) = _call(_stats_kernel, [e, wa, _b8(ba)],
                        [_BIG, _WSP, _SML],
                        [(_SML, ((8, _C), jnp.float32))], n)
        wb, bb_ = _fold(allr(st_a), *ws[1])
        x, e, st = _call(_fused_kernel,
                         [e, x, wa, _b8(ba), wb, _b8(bb_)],
                         [_BIG, _BIG, _WSP, _SML, _WSP, _SML], _XE_ST, n)
        st = allr(st)
    w2e, b2e = _fold(st, gamma2, beta2, W2, b2)
    (y,) = _call(_final_kernel, [e, x0f, w2e, _b8(b2e)],
                 [_BIG, _BIG, _WSP, _SML],
                 [(_BIG, (None, jnp.float32))], n)
    return y


def kernel(L, mask, inputs, W1, b1, rn, gamma2, beta2, W2, b2):
    del L, mask
    x0f = inputs.reshape(_N, _C)
    devs = jax.devices()
    ndev = 2 if len(devs) >= 2 and _N % (2 * _BLK) == 0 else 1
    if ndev == 1:
        y = _chain(x0f, W1, b1, rn, gamma2, beta2, W2, b2, _N, None)
        return y.reshape(1, _N, _C)
    mesh = Mesh(np.array(devs[:ndev]), ('d',))

    def shard_fn(x0f_s, W1_s, b1_s, rn_s, g2_s, bt2_s, W2_s, b2_s):
        return _chain(x0f_s, W1_s, b1_s, rn_s, g2_s, bt2_s, W2_s, b2_s,
                      _N // ndev, 'd')

    y = shard_map(
        shard_fn, mesh=mesh,
        in_specs=(P('d', None), P(), P(), P(), P(), P(), P(), P()),
        out_specs=P('d', None), check_rep=False,
    )(x0f, W1, b1, rn, gamma2, beta2, W2, b2)
    return y.reshape(1, _N, _C)


# speedup vs baseline: 2.1764x; 2.1764x over previous
import jax
import jax.numpy as jnp
from jax.experimental import pallas as pl

_N = 100000
_C = 128
_BLK = 20000
_EPS = 1e-5
_BF = jnp.bfloat16


def _elu(x):
    return jnp.where(x > 0, x, jnp.exp(x) - 1.0)


def _acc_stats(i, e32, st_ref):
    s = jnp.sum(e32, axis=0, keepdims=True)
    s2 = jnp.sum(e32 * e32, axis=0, keepdims=True)
    st = jnp.concatenate([s, s2, jnp.zeros((6, _C), jnp.float32)], axis=0)

    @pl.when(i == 0)
    def _init():
        st_ref[...] = jnp.zeros_like(st_ref)

    st_ref[...] += st


def _conv1_kernel(x_ref, w_ref, b_ref, xo_ref, eo_ref, st_ref):
    h = jnp.dot(x_ref[...].astype(_BF), w_ref[...],
                preferred_element_type=jnp.float32)
    h = h + b_ref[0:1, :]
    xo_ref[...] = h.astype(_BF)
    e = _elu(h)
    eo_ref[...] = e.astype(_BF)
    _acc_stats(pl.program_id(0), e, st_ref)


def _stats_kernel(e_ref, w_ref, b_ref, st_ref):
    h = jnp.dot(e_ref[...], w_ref[...], preferred_element_type=jnp.float32)
    h = h + b_ref[0:1, :]
    _acc_stats(pl.program_id(0), _elu(h), st_ref)


def _fused_kernel(e_ref, x_ref, wa_ref, ba_ref, wb_ref, bb_ref,
                  xo_ref, eo_ref, st_ref):
    ha = jnp.dot(e_ref[...], wa_ref[...], preferred_element_type=jnp.float32)
    ea = _elu(ha + ba_ref[0:1, :]).astype(_BF)
    hb = jnp.dot(ea, wb_ref[...], preferred_element_type=jnp.float32)
    hb = hb + bb_ref[0:1, :] + x_ref[...]
    xo_ref[...] = hb.astype(_BF)
    e = _elu(hb)
    eo_ref[...] = e.astype(_BF)
    _acc_stats(pl.program_id(0), e, st_ref)


def _final_kernel(e_ref, r_ref, w_ref, b_ref, y_ref):
    h = jnp.dot(e_ref[...], w_ref[...], preferred_element_type=jnp.float32)
    y_ref[...] = h + b_ref[0:1, :] + r_ref[...]


_BIG = pl.BlockSpec((_BLK, _C), lambda i: (i, 0))
_WSP = pl.BlockSpec((_C, _C), lambda i: (0, 0))
_SML = pl.BlockSpec((8, _C), lambda i: (0, 0))


def _call(kfn, ops, in_specs, outs):
    out_specs = [s for s, _ in outs]
    out_shape = [jax.ShapeDtypeStruct(sh, dt) for _, (sh, dt) in outs]
    return pl.pallas_call(kfn, grid=(_N // _BLK,), in_specs=in_specs,
                          out_specs=out_specs, out_shape=out_shape)(*ops)


def _b8(b):
    return jnp.broadcast_to(b[None, :], (8, _C))


def _fold(st, gamma_lo, beta_lo, w_lo, extra_b):
    m = st[0] * (1.0 / _N)
    v = st[1] * (1.0 / _N) - m * m
    scale = gamma_lo * jax.lax.rsqrt(v + _EPS)
    shift = beta_lo - m * scale
    return (scale[:, None] * w_lo).astype(_BF), shift @ w_lo + extra_b


_XE_ST = [(_BIG, ((_N, _C), _BF)), (_BIG, ((_N, _C), _BF)),
          (_SML, ((8, _C), jnp.float32))]


def kernel(L, mask, inputs, W1, b1, rn, gamma2, beta2, W2, b2):
    del L, mask
    x0f = inputs.reshape(_N, _C)
    x, e, st = _call(_conv1_kernel, [x0f, W1.astype(_BF), _b8(b1)],
                     [_BIG, _WSP, _SML], _XE_ST)
    for i in range(4):
        ws = []
        for h in range(2):
            g = rn['gamma%d' % h][i]
            bt = rn['beta%d' % h][i]
            W = rn['W%d' % h][i]
            bb = rn['b%d' % h][i]
            ws.append((g[:_C], bt[:_C], W[:_C], bt[_C:] @ W[_C:] + bb))
        wa, ba = _fold(st, *ws[0])
        (st_a,) = _call(_stats_kernel, [e, wa, _b8(ba)],
                        [_BIG, _WSP, _SML], [(_SML, ((8, _C), jnp.float32))])
        wb, bb_ = _fold(st_a, *ws[1])
        x, e, st = _call(_fused_kernel,
                         [e, x, wa, _b8(ba), wb, _b8(bb_)],
                         [_BIG, _BIG, _WSP, _SML, _WSP, _SML], _XE_ST)
    w2e, b2e = _fold(st, gamma2, beta2, W2, b2)
    (y,) = _call(_final_kernel, [e, x0f, w2e, _b8(b2e)],
                 [_BIG, _BIG, _WSP, _SML],
                 [(_BIG, ((_N, _C), jnp.float32))])
    return y.reshape(1, _N, _C)


# speedup vs baseline: 2.2799x; 1.0476x over previous
import jax
import jax.numpy as jnp
from jax.experimental import pallas as pl

_N = 100000
_C = 128
_BLK = 20000
_EPS = 1e-5
_BF = jnp.bfloat16


def _elu(x):
    return jnp.where(x > 0, x, jnp.exp(x) - 1.0)


def _acc_stats(i, e32, st_ref):
    s = jnp.sum(e32, axis=0, keepdims=True)
    s2 = jnp.sum(e32 * e32, axis=0, keepdims=True)
    st = jnp.concatenate([s, s2, jnp.zeros((6, _C), jnp.float32)], axis=0)

    @pl.when(i == 0)
    def _init():
        st_ref[...] = jnp.zeros_like(st_ref)

    st_ref[...] += st


def _conv1_kernel(x_ref, w_ref, b_ref, xo_ref, eo_ref, st_ref):
    h = jnp.dot(x_ref[...].astype(_BF), w_ref[...],
                preferred_element_type=jnp.float32)
    xb = (h + b_ref[0:1, :]).astype(_BF)
    xo_ref[...] = xb
    e = _elu(xb)
    eo_ref[...] = e
    _acc_stats(pl.program_id(0), e.astype(jnp.float32), st_ref)


def _stats_kernel(e_ref, w_ref, b_ref, st_ref):
    h = jnp.dot(e_ref[...], w_ref[...], preferred_element_type=jnp.float32)
    h = (h + b_ref[0:1, :]).astype(_BF)
    _acc_stats(pl.program_id(0), _elu(h).astype(jnp.float32), st_ref)


def _fused_kernel(e_ref, x_ref, wa_ref, ba_ref, wb_ref, bb_ref,
                  xo_ref, eo_ref, st_ref):
    ha = jnp.dot(e_ref[...], wa_ref[...], preferred_element_type=jnp.float32)
    ea = _elu((ha + ba_ref[0:1, :]).astype(_BF))
    hb = jnp.dot(ea, wb_ref[...], preferred_element_type=jnp.float32)
    xb = (hb + bb_ref[0:1, :] + x_ref[...]).astype(_BF)
    xo_ref[...] = xb
    e = _elu(xb)
    eo_ref[...] = e
    _acc_stats(pl.program_id(0), e.astype(jnp.float32), st_ref)


def _final_kernel(e_ref, r_ref, w_ref, b_ref, y_ref):
    h = jnp.dot(e_ref[...], w_ref[...], preferred_element_type=jnp.float32)
    y_ref[...] = h + b_ref[0:1, :] + r_ref[...]


_BIG = pl.BlockSpec((_BLK, _C), lambda i: (i, 0))
_WSP = pl.BlockSpec((_C, _C), lambda i: (0, 0))
_SML = pl.BlockSpec((8, _C), lambda i: (0, 0))


def _call(kfn, ops, in_specs, outs):
    out_specs = [s for s, _ in outs]
    out_shape = [jax.ShapeDtypeStruct(sh, dt) for _, (sh, dt) in outs]
    return pl.pallas_call(kfn, grid=(_N // _BLK,), in_specs=in_specs,
                          out_specs=out_specs, out_shape=out_shape)(*ops)


def _b8(b):
    return jnp.broadcast_to(b[None, :], (8, _C))


def _fold(st, gamma_lo, beta_lo, w_lo, extra_b):
    m = st[0] * (1.0 / _N)
    v = st[1] * (1.0 / _N) - m * m
    scale = gamma_lo * jax.lax.rsqrt(v + _EPS)
    shift = beta_lo - m * scale
    return (scale[:, None] * w_lo).astype(_BF), shift @ w_lo + extra_b


_XE_ST = [(_BIG, ((_N, _C), _BF)), (_BIG, ((_N, _C), _BF)),
          (_SML, ((8, _C), jnp.float32))]


def kernel(L, mask, inputs, W1, b1, rn, gamma2, beta2, W2, b2):
    del L, mask
    x0f = inputs.reshape(_N, _C)
    x, e, st = _call(_conv1_kernel, [x0f, W1.astype(_BF), _b8(b1)],
                     [_BIG, _WSP, _SML], _XE_ST)
    for i in range(4):
        ws = []
        for h in range(2):
            g = rn['gamma%d' % h][i]
            bt = rn['beta%d' % h][i]
            W = rn['W%d' % h][i]
            bb = rn['b%d' % h][i]
            ws.append((g[:_C], bt[:_C], W[:_C], bt[_C:] @ W[_C:] + bb))
        wa, ba = _fold(st, *ws[0])
        (st_a,) = _call(_stats_kernel, [e, wa, _b8(ba)],
                        [_BIG, _WSP, _SML], [(_SML, ((8, _C), jnp.float32))])
        wb, bb_ = _fold(st_a, *ws[1])
        x, e, st = _call(_fused_kernel,
                         [e, x, wa, _b8(ba), wb, _b8(bb_)],
                         [_BIG, _BIG, _WSP, _SML, _WSP, _SML], _XE_ST)
    w2e, b2e = _fold(st, gamma2, beta2, W2, b2)
    (y,) = _call(_final_kernel, [e, x0f, w2e, _b8(b2e)],
                 [_BIG, _BIG, _WSP, _SML],
                 [(_BIG, ((_N, _C), jnp.float32))])
    return y.reshape(1, _N, _C)
